# baseline (speedup 1.0000x reference)
import dataclasses
import functools

import jax
import jax.numpy as jnp
from jax import lax
from jax.experimental import pallas as pl
from jax.experimental.pallas import tpu as pltpu
from jax.experimental.pallas import tpu_sc as plsc

NC = 2
NS = 16
NW = NC * NS
CH = 64
K = 4


def _chunks(total, step=CH):
    out, o = [], 0
    while o < total:
        sz = min(step, total - o)
        out.append((o, sz))
        o += sz
    return out


def _make_seg_sum(n, d, e, with_cnt):
    epw = e // NW
    nfull = epw // CH
    rem = epw - nfull * CH
    assert epw * NW == e and rem % 16 == 0 and rem < CH
    assert nfull % K == 0 and nfull >= 3 * K
    rpt_a = -(-(n // NS) // 8) * 8
    rpt_b = n - (NS - 1) * rpt_a
    assert 0 < rpt_b <= rpt_a

    mesh = plsc.VectorSubcoreMesh(core_axis_name="c", subcore_axis_name="s")
    out_type = [jax.ShapeDtypeStruct((NC, n, d), jnp.float32)]
    scratch = (
        [pltpu.VMEM((2, CH), jnp.int32)] * K
        + [pltpu.VMEM((CH,), jnp.int32)] * K
        + [pltpu.VMEM((CH, d), jnp.float32)] * K
        + [pltpu.VMEM_SHARED((n, d), jnp.float32)]
        + [pltpu.SemaphoreType.DMA] * (3 * K)
    )
    if with_cnt:
        out_type.append(jax.ShapeDtypeStruct((NW, 1, n), jnp.float32))
        scratch.append(pltpu.VMEM((1, n), jnp.float32))
    if rem:
        scratch.append(pltpu.VMEM((2, rem), jnp.int32))

    cp = pltpu.CompilerParams()
    if "needs_layout_passes" in pltpu.CompilerParams.__dataclass_fields__:
        cp = dataclasses.replace(cp, needs_layout_passes=False)

    @functools.partial(pl.kernel, mesh=mesh, out_type=out_type,
                       scratch_types=scratch, compiler_params=cp)
    def seg_sum(x_hbm, idx_hbm, rem_hbm, agg_hbm, *rest):
        rest = list(rest)
        cnt_hbm = rest.pop(0) if with_cnt else None
        ibufs = tuple(rest[:K])
        didx_b = tuple(rest[K:2 * K])
        rows_b = tuple(rest[2 * K:3 * K])
        agg_sp = rest[3 * K]
        isems = tuple(rest[3 * K + 1:4 * K + 1])
        gsems = tuple(rest[4 * K + 1:5 * K + 1])
        ssems = tuple(rest[5 * K + 1:6 * K + 1])
        rows0 = rows_b[0]
        rest = rest[6 * K + 1:]
        if with_cnt:
            cnt_v = rest.pop(0)
        if rem:
            (rbuf,) = rest

        c = lax.axis_index("c")
        s = lax.axis_index("s")
        w = c * NS + s
        z16 = jnp.zeros((16,), jnp.float32)
        z16i = jnp.zeros((16,), jnp.int32)
        one16 = jnp.full((16,), 1.0, jnp.float32)

        def i_start(i, slot):
            pltpu.async_copy(idx_hbm.at[w, i], ibufs[slot], isems[slot])

        def i_wait(i, slot):
            pltpu.make_async_copy(idx_hbm.at[w, i], ibufs[slot],
                                  isems[slot]).wait()

        def g_start(slot):
            pltpu.async_copy(x_hbm.at[ibufs[slot].at[0]], rows_b[slot],
                             gsems[slot])

        def g_wait(slot):
            pltpu.make_async_copy(x_hbm.at[ibufs[slot].at[0]], rows_b[slot],
                                  gsems[slot]).wait()

        def didx_copy(slot):
            for kk in range(CH // 16):
                didx_b[slot][pl.ds(kk * 16, 16)] = (
                    ibufs[slot][1, pl.ds(kk * 16, 16)])

        def s_start(slot):
            pltpu.async_copy(rows_b[slot], agg_sp.at[didx_b[slot]],
                             ssems[slot], add=True)

        def s_wait(slot):
            pltpu.make_async_copy(rows_b[slot], agg_sp.at[didx_b[slot]],
                                  ssems[slot]).wait()

        def cnt_upd(slot):
            if with_cnt:
                for kk in range(CH // 16):
                    idx16 = didx_b[slot][pl.ds(kk * 16, 16)]
                    plsc.addupdate_scatter(cnt_v, [z16i, idx16], one16)

        for t in range(K):
            i_start(t, t)

        @pl.loop(0, CH)
        def _(r):
            for cc in range(d // 16):
                rows0[r, pl.ds(cc * 16, 16)] = z16

        if with_cnt:
            @pl.loop(0, n // 16)
            def _(i):
                cnt_v[0, pl.ds(i * 16, 16)] = z16

        def zero_fill(base, size):
            for (o, sz) in _chunks(size):
                pltpu.sync_copy(rows0.at[pl.ds(0, sz)],
                                agg_sp.at[pl.ds(base + o, sz)])

        def flush(base, size):
            pltpu.sync_copy(agg_sp.at[pl.ds(base, size)],
                            agg_hbm.at[c].at[pl.ds(base, size)])

        base_a = pl.multiple_of(s * rpt_a, 8)

        @pl.when(s < NS - 1)
        def _():
            zero_fill(base_a, rpt_a)

        @pl.when(s == NS - 1)
        def _():
            zero_fill((NS - 1) * rpt_a, rpt_b)

        plsc.subcore_barrier()

        def body(i, slot, start_idx, start_gather, first=False):
            sp = (slot + K - 1) % K
            g_wait(slot)
            if not first:
                s_wait(sp)
            if start_gather:
                i_wait(i + K - 1, sp)
                g_start(sp)
            didx_copy(slot)
            if start_idx:
                i_start(i + K, slot)
            s_start(slot)
            cnt_upd(slot)

        for t in range(K - 1):
            i_wait(t, t)
            g_start(t)
        body(0, 0, True, True, first=True)

        @pl.loop(0, (nfull - 2 * K) // K)
        def _(j):
            i0 = K * j + 1
            for t in range(K):
                body(i0 + t, (t + 1) % K, True, True)

        for i in range(nfull - 2 * K + 1, nfull):
            body(i, i % K, i <= nfull - K - 1, i <= nfull - K)
        s_wait((nfull - 1) % K)

        if rem:
            pltpu.sync_copy(rem_hbm.at[w], rbuf)
            pltpu.sync_copy(x_hbm.at[rbuf.at[0]], rows0.at[pl.ds(0, rem)])
            pltpu.sync_copy(rows0.at[pl.ds(0, rem)],
                            agg_sp.at[rbuf.at[1]], add=True)
            if with_cnt:
                for kk in range(rem // 16):
                    idx16 = rbuf[1, pl.ds(kk * 16, 16)]
                    plsc.addupdate_scatter(cnt_v, [z16i, idx16], one16)

        plsc.subcore_barrier()

        @pl.when(s < NS - 1)
        def _():
            flush(base_a, rpt_a)

        @pl.when(s == NS - 1)
        def _():
            flush((NS - 1) * rpt_a, rpt_b)

        if with_cnt:
            pltpu.sync_copy(cnt_v, cnt_hbm.at[w])

    return seg_sum


def _dot_t(a, w):
    return lax.dot_general(a, w, (((1,), (1,)), ((), ())),
                           preferred_element_type=jnp.float32)


def _tc1_body(aggp, cntp, x, wl, bl, wr, gamma, beta, h_out, invc_out):
    cnt = jnp.sum(cntp[...], axis=1, keepdims=True)
    invc = 1.0 / jnp.maximum(cnt, 1.0)
    mean_agg = (aggp[0] + aggp[1]) * invc
    h = _dot_t(mean_agg, wl[...]) + bl[...][None, :] + _dot_t(x[...], wr[...])
    mu = jnp.mean(h, axis=0, keepdims=True)
    hc = h - mu
    var = jnp.mean(hc * hc, axis=0, keepdims=True)
    hn = hc / jnp.sqrt(var + 1e-5) * gamma[...][None, :] + beta[...][None, :]
    h_out[...] = jnp.maximum(hn, 0.0)
    invc_out[...] = invc


def _tc2_body(aggp, invc, h, wl, bl, wr, out):
    mean_agg = (aggp[0] + aggp[1]) * invc[...]
    out[...] = (_dot_t(mean_agg, wl[...]) + bl[...][None, :]
                + _dot_t(h[...], wr[...]))


def kernel(x, edge_index, Wl1, bl1, Wr1, gamma, beta, Wl2, bl2, Wr2):
    n, d = x.shape
    e = edge_index.shape[1]
    src = edge_index[0].astype(jnp.int32)
    dst = edge_index[1].astype(jnp.int32)

    epw = e // NW
    nfull = epw // CH
    rem = epw - nfull * CH
    srcw = src.reshape(NW, epw)
    dstw = dst.reshape(NW, epw)
    idx_blocks = jnp.stack(
        [srcw[:, :nfull * CH].reshape(NW, nfull, CH),
         dstw[:, :nfull * CH].reshape(NW, nfull, CH)], axis=2)
    rem_blocks = jnp.stack([srcw[:, nfull * CH:], dstw[:, nfull * CH:]],
                           axis=1)

    seg_sum_cnt = _make_seg_sum(n, d, e, with_cnt=True)
    seg_sum = _make_seg_sum(n, d, e, with_cnt=False)

    agg1, cntp = seg_sum_cnt(x, idx_blocks, rem_blocks)
    cnt_t = cntp.reshape(NW, n).T

    h, invc = pl.pallas_call(
        _tc1_body,
        out_shape=[jax.ShapeDtypeStruct((n, d), jnp.float32),
                   jax.ShapeDtypeStruct((n, 1), jnp.float32)],
    )(agg1, cnt_t, x, Wl1, bl1, Wr1, gamma, beta)

    (agg2,) = seg_sum(h, idx_blocks, rem_blocks)

    out = pl.pallas_call(
        _tc2_body,
        out_shape=jax.ShapeDtypeStruct((n, d), jnp.float32),
    )(agg2, invc, h, Wl2, bl2, Wr2)
    return out


# speedup vs baseline: 1.1197x; 1.1197x over previous
import dataclasses
import functools

import jax
import jax.numpy as jnp
from jax import lax
from jax.experimental import pallas as pl
from jax.experimental.pallas import tpu as pltpu
from jax.experimental.pallas import tpu_sc as plsc

NC = 2
NS = 16
NW = NC * NS
CH = 96


def _chunks(total, step=CH):
    out, o = [], 0
    while o < total:
        sz = min(step, total - o)
        out.append((o, sz))
        o += sz
    return out


def _make_seg_sum(n, d, e, with_cnt):
    epw = e // NW
    nfull = epw // CH
    rem = epw - nfull * CH
    assert epw * NW == e and rem % 16 == 0 and rem < CH
    assert nfull >= 8 and (nfull - 5) % 3 == 0
    rpt_a = -(-(n // NS) // 8) * 8
    rpt_b = n - (NS - 1) * rpt_a
    assert 0 < rpt_b <= rpt_a

    mesh = plsc.VectorSubcoreMesh(core_axis_name="c", subcore_axis_name="s")
    out_type = [jax.ShapeDtypeStruct((NC, n, d), jnp.float32)]
    scratch = (
        [pltpu.VMEM((2, CH), jnp.int32)] * 3
        + [pltpu.VMEM((CH,), jnp.int32)] * 3
        + [pltpu.VMEM((CH, d), jnp.float32)] * 3
        + [pltpu.VMEM_SHARED((n, d), jnp.float32)]
        + [pltpu.SemaphoreType.DMA] * 9
    )
    if with_cnt:
        out_type.append(jax.ShapeDtypeStruct((NW, 1, n), jnp.float32))
        scratch.append(pltpu.VMEM((1, n), jnp.float32))
    if rem:
        scratch.append(pltpu.VMEM((2, rem), jnp.int32))

    cp = pltpu.CompilerParams()
    if "needs_layout_passes" in pltpu.CompilerParams.__dataclass_fields__:
        cp = dataclasses.replace(cp, needs_layout_passes=False)

    @functools.partial(pl.kernel, mesh=mesh, out_type=out_type,
                       scratch_types=scratch, compiler_params=cp)
    def seg_sum(x_hbm, idx_hbm, rem_hbm, agg_hbm, *rest):
        rest = list(rest)
        cnt_hbm = rest.pop(0) if with_cnt else None
        ibufs = tuple(rest[:3])
        didx_b = tuple(rest[3:6])
        rows_b = tuple(rest[6:9])
        agg_sp = rest[9]
        isems = tuple(rest[10:13])
        gsems = tuple(rest[13:16])
        ssems = tuple(rest[16:19])
        rows0 = rows_b[0]
        rest = rest[19:]
        if with_cnt:
            cnt_v = rest.pop(0)
        if rem:
            (rbuf,) = rest

        c = lax.axis_index("c")
        s = lax.axis_index("s")
        w = c * NS + s
        z16 = jnp.zeros((16,), jnp.float32)
        z16i = jnp.zeros((16,), jnp.int32)
        one16 = jnp.full((16,), 1.0, jnp.float32)

        def i_start(i, slot):
            pltpu.async_copy(idx_hbm.at[w, i], ibufs[slot], isems[slot])

        def i_wait(i, slot):
            pltpu.make_async_copy(idx_hbm.at[w, i], ibufs[slot],
                                  isems[slot]).wait()

        def g_start(slot):
            pltpu.async_copy(x_hbm.at[ibufs[slot].at[0]], rows_b[slot],
                             gsems[slot])

        def g_wait(slot):
            pltpu.make_async_copy(x_hbm.at[ibufs[slot].at[0]], rows_b[slot],
                                  gsems[slot]).wait()

        def didx_copy(slot):
            for kk in range(CH // 16):
                didx_b[slot][pl.ds(kk * 16, 16)] = (
                    ibufs[slot][1, pl.ds(kk * 16, 16)])

        def s_start(slot):
            pltpu.async_copy(rows_b[slot], agg_sp.at[didx_b[slot]],
                             ssems[slot], add=True)

        def s_wait(slot):
            pltpu.make_async_copy(rows_b[slot], agg_sp.at[didx_b[slot]],
                                  ssems[slot]).wait()

        def cnt_upd(slot):
            if with_cnt:
                for kk in range(CH // 16):
                    idx16 = didx_b[slot][pl.ds(kk * 16, 16)]
                    plsc.addupdate_scatter(cnt_v, [z16i, idx16], one16)

        i_start(0, 0)
        i_start(1, 1)
        i_start(2, 2)

        @pl.loop(0, CH)
        def _(r):
            for cc in range(d // 16):
                rows0[r, pl.ds(cc * 16, 16)] = z16

        if with_cnt:
            @pl.loop(0, n // 16)
            def _(i):
                cnt_v[0, pl.ds(i * 16, 16)] = z16

        def zero_fill(base, size):
            for (o, sz) in _chunks(size):
                pltpu.sync_copy(rows0.at[pl.ds(0, sz)],
                                agg_sp.at[pl.ds(base + o, sz)])

        def flush(base, size):
            pltpu.sync_copy(agg_sp.at[pl.ds(base, size)],
                            agg_hbm.at[c].at[pl.ds(base, size)])

        base_a = pl.multiple_of(s * rpt_a, 8)

        @pl.when(s < NS - 1)
        def _():
            zero_fill(base_a, rpt_a)

        @pl.when(s == NS - 1)
        def _():
            zero_fill((NS - 1) * rpt_a, rpt_b)

        plsc.subcore_barrier()

        def body(i, slot, start_idx, start_gather, first=False):
            sp = (slot + 2) % 3
            g_wait(slot)
            if not first:
                s_wait(sp)
            if start_gather:
                i_wait(i + 2, sp)
                g_start(sp)
            didx_copy(slot)
            if start_idx:
                i_start(i + 3, slot)
            s_start(slot)
            cnt_upd(slot)

        i_wait(0, 0)
        g_start(0)
        i_wait(1, 1)
        g_start(1)
        body(0, 0, True, True, first=True)

        @pl.loop(0, (nfull - 5) // 3)
        def _(j):
            i0 = 3 * j + 1
            body(i0, 1, True, True)
            body(i0 + 1, 2, True, True)
            body(i0 + 2, 0, True, True)

        body(nfull - 4, 1, True, True)
        body(nfull - 3, 2, False, True)
        body(nfull - 2, 0, False, False)
        body(nfull - 1, 1, False, False)
        s_wait(1)

        if rem:
            pltpu.sync_copy(rem_hbm.at[w], rbuf)
            pltpu.sync_copy(x_hbm.at[rbuf.at[0]], rows0.at[pl.ds(0, rem)])
            pltpu.sync_copy(rows0.at[pl.ds(0, rem)],
                            agg_sp.at[rbuf.at[1]], add=True)
            if with_cnt:
                for kk in range(rem // 16):
                    idx16 = rbuf[1, pl.ds(kk * 16, 16)]
                    plsc.addupdate_scatter(cnt_v, [z16i, idx16], one16)

        plsc.subcore_barrier()

        @pl.when(s < NS - 1)
        def _():
            flush(base_a, rpt_a)

        @pl.when(s == NS - 1)
        def _():
            flush((NS - 1) * rpt_a, rpt_b)

        if with_cnt:
            pltpu.sync_copy(cnt_v, cnt_hbm.at[w])

    return seg_sum


def _dot_t(a, w):
    return lax.dot_general(a, w, (((1,), (1,)), ((), ())),
                           preferred_element_type=jnp.float32)


def _tc1_body(aggp, cntp, x, wl, bl, wr, gamma, beta, h_out, invc_out):
    cnt = jnp.sum(cntp[...], axis=1, keepdims=True)
    invc = 1.0 / jnp.maximum(cnt, 1.0)
    mean_agg = (aggp[0] + aggp[1]) * invc
    h = _dot_t(mean_agg, wl[...]) + bl[...][None, :] + _dot_t(x[...], wr[...])
    mu = jnp.mean(h, axis=0, keepdims=True)
    hc = h - mu
    var = jnp.mean(hc * hc, axis=0, keepdims=True)
    hn = hc / jnp.sqrt(var + 1e-5) * gamma[...][None, :] + beta[...][None, :]
    h_out[...] = jnp.maximum(hn, 0.0)
    invc_out[...] = invc


def _tc2_body(aggp, invc, h, wl, bl, wr, out):
    mean_agg = (aggp[0] + aggp[1]) * invc[...]
    out[...] = (_dot_t(mean_agg, wl[...]) + bl[...][None, :]
                + _dot_t(h[...], wr[...]))


def kernel(x, edge_index, Wl1, bl1, Wr1, gamma, beta, Wl2, bl2, Wr2):
    n, d = x.shape
    e = edge_index.shape[1]
    src = edge_index[0].astype(jnp.int32)
    dst = edge_index[1].astype(jnp.int32)

    epw = e // NW
    nfull = epw // CH
    rem = epw - nfull * CH
    srcw = src.reshape(NW, epw)
    dstw = dst.reshape(NW, epw)
    idx_blocks = jnp.stack(
        [srcw[:, :nfull * CH].reshape(NW, nfull, CH),
         dstw[:, :nfull * CH].reshape(NW, nfull, CH)], axis=2)
    rem_blocks = jnp.stack([srcw[:, nfull * CH:], dstw[:, nfull * CH:]],
                           axis=1)

    seg_sum_cnt = _make_seg_sum(n, d, e, with_cnt=True)
    seg_sum = _make_seg_sum(n, d, e, with_cnt=False)

    agg1, cntp = seg_sum_cnt(x, idx_blocks, rem_blocks)
    cnt_t = cntp.reshape(NW, n).T

    h, invc = pl.pallas_call(
        _tc1_body,
        out_shape=[jax.ShapeDtypeStruct((n, d), jnp.float32),
                   jax.ShapeDtypeStruct((n, 1), jnp.float32)],
    )(agg1, cnt_t, x, Wl1, bl1, Wr1, gamma, beta)

    (agg2,) = seg_sum(h, idx_blocks, rem_blocks)

    out = pl.pallas_call(
        _tc2_body,
        out_shape=jax.ShapeDtypeStruct((n, d), jnp.float32),
    )(agg2, invc, h, Wl2, bl2, Wr2)
    return out
